# Initial kernel scaffold; baseline (speedup 1.0000x reference)
import functools

import jax
import jax.numpy as jnp
from jax import lax
from jax.experimental import pallas as pl
from jax.experimental.pallas import tpu as pltpu
from jax.experimental.pallas import tpu_sc as plsc

N = 10000
E = 320000
D = 128
AUG = 16
DW = D + AUG

NC = 2
NS = 16
NW = NC * NS
K = 80
RPT = N // NS

_mesh = plsc.VectorSubcoreMesh(
    core_axis_name="c", subcore_axis_name="s", num_cores=NC, num_subcores=NS)


@functools.partial(
    pl.kernel,
    out_type=jax.ShapeDtypeStruct((NC, N, DW), jnp.float32),
    mesh=_mesh,
    scratch_types=[
        pltpu.VMEM_SHARED((N, DW), jnp.float32),
        pltpu.VMEM((K,), jnp.int32),
        pltpu.VMEM((K,), jnp.int32),
        pltpu.VMEM((K, DW), jnp.float32),
        pltpu.SemaphoreType.DMA,
    ],
)
def _sc_aggr(xaug, src_hbm, dst_hbm, zeros_hbm, out, acc, idx_g, idx_s, rows,
             sem):
    c = lax.axis_index("c")
    s = lax.axis_index("s")
    wid = c * NS + s
    pltpu.sync_copy(zeros_hbm, acc.at[pl.ds(s * RPT, RPT)])
    plsc.subcore_barrier()
    epw = E // NW
    ebase = wid * epw

    def step(k, carry):
        off = ebase + k * K
        pltpu.sync_copy(src_hbm.at[pl.ds(off, K)], idx_g)
        pltpu.sync_copy(dst_hbm.at[pl.ds(off, K)], idx_s)
        pltpu.async_copy(xaug.at[idx_g], rows, sem).wait()
        pltpu.sync_copy(rows, acc.at[idx_s], add=True)
        return carry

    lax.fori_loop(0, epw // K, step, 0)
    plsc.subcore_barrier()
    pltpu.sync_copy(acc.at[pl.ds(s * RPT, RPT)],
                    out.at[c, pl.ds(s * RPT, RPT)])


@functools.partial(
    pl.kernel,
    out_type=jax.ShapeDtypeStruct((NC, N, DW), jnp.float32),
    mesh=_mesh,
    scratch_types=[
        pltpu.VMEM_SHARED((N, DW), jnp.float32),
        pltpu.VMEM((K,), jnp.int32),
        pltpu.VMEM((K,), jnp.int32),
        pltpu.VMEM((K, DW), jnp.float32),
        pltpu.SemaphoreType.DMA,
    ],
)
def _sc_gate(baug, b2aug, src_hbm, dst_hbm, zeros_hbm, out, acc, idx_g, idx_s,
             rows, sem):
    c = lax.axis_index("c")
    s = lax.axis_index("s")
    pltpu.sync_copy(zeros_hbm, acc.at[pl.ds(s * RPT, RPT)])
    plsc.subcore_barrier()
    eps = E // NS
    ebase = s * eps

    def make_step(table):
        def step(k, carry):
            off = ebase + k * K
            pltpu.sync_copy(dst_hbm.at[pl.ds(off, K)], idx_g)
            pltpu.sync_copy(src_hbm.at[pl.ds(off, K)], idx_s)
            pltpu.async_copy(table.at[idx_g], rows, sem).wait()
            pltpu.sync_copy(rows, acc.at[idx_s], add=True)
            return carry
        return step

    @pl.when(c == 0)
    def _():
        lax.fori_loop(0, eps // K, make_step(baug), 0)

    @pl.when(c == 1)
    def _():
        lax.fori_loop(0, eps // K, make_step(b2aug), 0)

    plsc.subcore_barrier()
    pltpu.sync_copy(acc.at[pl.ds(s * RPT, RPT)],
                    out.at[c, pl.ds(s * RPT, RPT)])


_BN = 1000


def _tc_mid_body(x_ref, ap_ref, wl_ref, bl_ref, wr_ref, qw_ref, qb_ref,
                 a_ref, baug_ref, b2aug_ref):
    ap = ap_ref[...]
    agg = ap[0] + ap[1]
    deg = jnp.maximum(agg[:, D:D + 1], 1.0)
    aggr = agg[:, :D] / deg
    x = x_ref[...]
    wl = wl_ref[...]
    wr = wr_ref[...]
    qw = qw_ref[...]
    dn = (((1,), (1,)), ((), ()))
    xc = lax.dot_general(aggr, wl, dn, preferred_element_type=jnp.float32)
    xc = xc + lax.dot_general(x, wr, dn, preferred_element_type=jnp.float32)
    xc = jnp.maximum(xc + bl_ref[...], 0.0)
    a = lax.dot_general(xc, qw[:, :D], dn, preferred_element_type=jnp.float32)
    b = lax.dot_general(xc, qw[:, D:], dn, preferred_element_type=jnp.float32)
    b = b + qb_ref[...]
    a_ref[...] = a
    ones = jnp.ones((_BN, AUG), jnp.float32)
    baug_ref[...] = jnp.concatenate([b, ones], axis=1)
    b2aug_ref[...] = jnp.concatenate([b * b, ones], axis=1)


def _tc_mid(x, aggr_part, wl, bl, wr, qw, qb):
    grid = (N // _BN,)
    return pl.pallas_call(
        _tc_mid_body,
        grid=grid,
        in_specs=[
            pl.BlockSpec((_BN, D), lambda i: (i, 0)),
            pl.BlockSpec((NC, _BN, DW), lambda i: (0, i, 0)),
            pl.BlockSpec((D, D), lambda i: (0, 0)),
            pl.BlockSpec((1, D), lambda i: (0, 0)),
            pl.BlockSpec((D, D), lambda i: (0, 0)),
            pl.BlockSpec((D, 2 * D), lambda i: (0, 0)),
            pl.BlockSpec((1, D), lambda i: (0, 0)),
        ],
        out_specs=[
            pl.BlockSpec((_BN, D), lambda i: (i, 0)),
            pl.BlockSpec((_BN, DW), lambda i: (i, 0)),
            pl.BlockSpec((_BN, DW), lambda i: (i, 0)),
        ],
        out_shape=[
            jax.ShapeDtypeStruct((N, D), jnp.float32),
            jax.ShapeDtypeStruct((N, DW), jnp.float32),
            jax.ShapeDtypeStruct((N, DW), jnp.float32),
        ],
    )(x, aggr_part, wl, bl, wr, qw, qb)


def _tc_out_body(a_ref, sp_ref, gg_ref):
    sp = sp_ref[...]
    sb = sp[0, :, :D]
    sb2 = sp[1, :, :D]
    cnt = sp[0, :, D:D + 1]
    a = a_ref[...]
    mean_q = (cnt * a * a + 2.0 * a * sb + sb2) / jnp.maximum(cnt, 1.0)
    gg_ref[...] = jnp.tanh(mean_q)


def _tc_out(a, sb_part):
    grid = (N // _BN,)
    return pl.pallas_call(
        _tc_out_body,
        grid=grid,
        in_specs=[
            pl.BlockSpec((_BN, D), lambda i: (i, 0)),
            pl.BlockSpec((NC, _BN, DW), lambda i: (0, i, 0)),
        ],
        out_specs=pl.BlockSpec((_BN, D), lambda i: (i, 0)),
        out_shape=jax.ShapeDtypeStruct((N, D), jnp.float32),
    )(a, sb_part)


def kernel(X, edge_index, Wl, bl, Wr, Qw, Qb):
    src = edge_index[0]
    dst = edge_index[1]
    ones = jnp.ones((N, AUG), jnp.float32)
    xaug = jnp.concatenate([X, ones], axis=1)
    zeros = jnp.zeros((RPT, DW), jnp.float32)

    aggr_part = _sc_aggr(xaug, src, dst, zeros)
    a, baug, b2aug = _tc_mid(X, aggr_part, Wl, bl.reshape(1, D), Wr, Qw,
                             Qb.reshape(1, D))
    sb_part = _sc_gate(baug, b2aug, src, dst, zeros)
    return _tc_out(a, sb_part)


# speedup vs baseline: 4.3469x; 4.3469x over previous
import functools

import jax
import jax.numpy as jnp
from jax import lax
from jax.experimental import pallas as pl
from jax.experimental.pallas import tpu as pltpu
from jax.experimental.pallas import tpu_sc as plsc

N = 10000
E = 320000
D = 128

NC = 2
NS = 16
NW = NC * NS
K = 80
L = 16
NP = 10240
RPT = NP // NS

_mesh = plsc.VectorSubcoreMesh(
    core_axis_name="c", subcore_axis_name="s", num_cores=NC, num_subcores=NS)


@functools.partial(
    pl.kernel,
    out_type=(
        jax.ShapeDtypeStruct((NC, NP, D), jnp.float32),
        jax.ShapeDtypeStruct((NW, NP), jnp.float32),
        jax.ShapeDtypeStruct((NW, NP), jnp.float32),
    ),
    mesh=_mesh,
    compiler_params=pltpu.CompilerParams(needs_layout_passes=False),
    scratch_types=[
        pltpu.VMEM_SHARED((NP, D), jnp.float32),
        pltpu.VMEM((K,), jnp.int32),
        pltpu.VMEM((K,), jnp.int32),
        pltpu.VMEM((K, D), jnp.float32),
        pltpu.VMEM((NP,), jnp.float32),
        pltpu.VMEM((NP,), jnp.float32),
        pltpu.SemaphoreType.DMA,
    ],
)
def _sc_aggr(x_hbm, src_hbm, dst_hbm, zrows_hbm, zflat_hbm,
             out, deg_out, cnt_out,
             acc, idx_g, idx_s, rows, deg_l, cnt_l, sem):
    c = lax.axis_index("c")
    s = lax.axis_index("s")
    wid = c * NS + s
    pltpu.sync_copy(zrows_hbm, acc.at[pl.ds(s * RPT, RPT)])
    pltpu.sync_copy(zflat_hbm, deg_l)
    pltpu.sync_copy(zflat_hbm, cnt_l)
    plsc.subcore_barrier()
    epw = E // NW
    ebase = wid * epw
    ones = jnp.ones((L,), jnp.float32)

    def step(k, carry):
        off = ebase + k * K
        pltpu.sync_copy(src_hbm.at[pl.ds(off, K)], idx_g)
        pltpu.sync_copy(dst_hbm.at[pl.ds(off, K)], idx_s)
        gather = pltpu.async_copy(x_hbm.at[idx_g], rows, sem)
        for j in range(K // L):
            plsc.addupdate_scatter(cnt_l, [idx_g[pl.ds(j * L, L)]], ones)
            plsc.addupdate_scatter(deg_l, [idx_s[pl.ds(j * L, L)]], ones)
        gather.wait()
        pltpu.sync_copy(rows, acc.at[idx_s], add=True)
        return carry

    lax.fori_loop(0, epw // K, step, 0)
    pltpu.sync_copy(deg_l, deg_out.at[wid])
    pltpu.sync_copy(cnt_l, cnt_out.at[wid])
    plsc.subcore_barrier()
    pltpu.sync_copy(acc.at[pl.ds(s * RPT, RPT)],
                    out.at[c, pl.ds(s * RPT, RPT)])


@functools.partial(
    pl.kernel,
    out_type=jax.ShapeDtypeStruct((NC, NP, D), jnp.float32),
    mesh=_mesh,
    compiler_params=pltpu.CompilerParams(needs_layout_passes=False),
    scratch_types=[
        pltpu.VMEM_SHARED((NP, D), jnp.float32),
        pltpu.VMEM((K,), jnp.int32),
        pltpu.VMEM((K,), jnp.int32),
        pltpu.VMEM((K, D), jnp.float32),
        pltpu.SemaphoreType.DMA,
    ],
)
def _sc_gate(b_hbm, b2_hbm, src_hbm, dst_hbm, zrows_hbm, out,
             acc, idx_g, idx_s, rows, sem):
    c = lax.axis_index("c")
    s = lax.axis_index("s")
    pltpu.sync_copy(zrows_hbm, acc.at[pl.ds(s * RPT, RPT)])
    plsc.subcore_barrier()
    eps = E // NS
    ebase = s * eps

    def make_step(table):
        def step(k, carry):
            off = ebase + k * K
            pltpu.sync_copy(dst_hbm.at[pl.ds(off, K)], idx_g)
            pltpu.sync_copy(src_hbm.at[pl.ds(off, K)], idx_s)
            pltpu.async_copy(table.at[idx_g], rows, sem).wait()
            pltpu.sync_copy(rows, acc.at[idx_s], add=True)
            return carry
        return step

    @pl.when(c == 0)
    def _():
        lax.fori_loop(0, eps // K, make_step(b_hbm), 0)

    @pl.when(c == 1)
    def _():
        lax.fori_loop(0, eps // K, make_step(b2_hbm), 0)

    plsc.subcore_barrier()
    pltpu.sync_copy(acc.at[pl.ds(s * RPT, RPT)],
                    out.at[c, pl.ds(s * RPT, RPT)])


_BN = 1024


def _tc_mid_body(x_ref, ap_ref, dp_ref, wl_ref, bl_ref, wr_ref, qw_ref,
                 qb_ref, a_ref, b_ref, b2_ref):
    ap = ap_ref[...]
    agg = ap[0] + ap[1]
    deg = jnp.maximum(jnp.sum(dp_ref[...], axis=0), 1.0)[:, None]
    aggr = agg / deg
    x = x_ref[...]
    wl = wl_ref[...]
    wr = wr_ref[...]
    qw = qw_ref[...]
    dn = (((1,), (1,)), ((), ()))
    xc = lax.dot_general(aggr, wl, dn, preferred_element_type=jnp.float32)
    xc = xc + lax.dot_general(x, wr, dn, preferred_element_type=jnp.float32)
    xc = jnp.maximum(xc + bl_ref[...], 0.0)
    a = lax.dot_general(xc, qw[:, :D], dn, preferred_element_type=jnp.float32)
    b = lax.dot_general(xc, qw[:, D:], dn, preferred_element_type=jnp.float32)
    b = b + qb_ref[...]
    a_ref[...] = a
    b_ref[...] = b
    b2_ref[...] = b * b


def _tc_mid(x, aggr_part, deg_part, wl, bl, wr, qw, qb):
    grid = (NP // _BN,)
    return pl.pallas_call(
        _tc_mid_body,
        grid=grid,
        in_specs=[
            pl.BlockSpec((_BN, D), lambda i: (i, 0)),
            pl.BlockSpec((NC, _BN, D), lambda i: (0, i, 0)),
            pl.BlockSpec((NW, _BN), lambda i: (0, i)),
            pl.BlockSpec((D, D), lambda i: (0, 0)),
            pl.BlockSpec((1, D), lambda i: (0, 0)),
            pl.BlockSpec((D, D), lambda i: (0, 0)),
            pl.BlockSpec((D, 2 * D), lambda i: (0, 0)),
            pl.BlockSpec((1, D), lambda i: (0, 0)),
        ],
        out_specs=[
            pl.BlockSpec((_BN, D), lambda i: (i, 0)),
            pl.BlockSpec((_BN, D), lambda i: (i, 0)),
            pl.BlockSpec((_BN, D), lambda i: (i, 0)),
        ],
        out_shape=[
            jax.ShapeDtypeStruct((N, D), jnp.float32),
            jax.ShapeDtypeStruct((N, D), jnp.float32),
            jax.ShapeDtypeStruct((N, D), jnp.float32),
        ],
    )(x, aggr_part, deg_part, wl, bl, wr, qw, qb)


def _tc_out_body(a_ref, sp_ref, cp_ref, gg_ref):
    sp = sp_ref[...]
    sb = sp[0]
    sb2 = sp[1]
    cnt = jnp.sum(cp_ref[...], axis=0)[:, None]
    a = a_ref[...]
    mean_q = (cnt * a * a + 2.0 * a * sb + sb2) / jnp.maximum(cnt, 1.0)
    gg_ref[...] = jnp.tanh(mean_q)


def _tc_out(a, sb_part, cnt_part):
    grid = (NP // _BN,)
    return pl.pallas_call(
        _tc_out_body,
        grid=grid,
        in_specs=[
            pl.BlockSpec((_BN, D), lambda i: (i, 0)),
            pl.BlockSpec((NC, _BN, D), lambda i: (0, i, 0)),
            pl.BlockSpec((NW, _BN), lambda i: (0, i)),
        ],
        out_specs=pl.BlockSpec((_BN, D), lambda i: (i, 0)),
        out_shape=jax.ShapeDtypeStruct((N, D), jnp.float32),
    )(a, sb_part, cnt_part)


def kernel(X, edge_index, Wl, bl, Wr, Qw, Qb):
    src = edge_index[0]
    dst = edge_index[1]
    zrows = jnp.zeros((RPT, D), jnp.float32)
    zflat = jnp.zeros((NP,), jnp.float32)

    aggr_part, deg_part, cnt_part = _sc_aggr(X, src, dst, zrows, zflat)
    a, b, b2 = _tc_mid(X, aggr_part, deg_part, Wl, bl.reshape(1, D), Wr, Qw,
                       Qb.reshape(1, D))
    sb_part = _sc_gate(b, b2, src, dst, zrows)
    return _tc_out(a, sb_part, cnt_part)
